# Initial kernel scaffold; baseline (speedup 1.0000x reference)
import jax
import jax.numpy as jnp
from jax.experimental import pallas as pl


def kernel(x, edge_index, edge_attr, W1, b1, W2, b2):
    raise NotImplementedError("write your pallas kernel here")


# speedup vs baseline: 3.3702x; 3.3702x over previous
import functools

import jax
import jax.numpy as jnp
from jax import lax
from jax.experimental import pallas as pl
from jax.experimental.pallas import tpu as pltpu
from jax.experimental.pallas import tpu_sc as plsc

N = 10000
E = 320000
D = 128

NC = 2
NS = 16
NW = NC * NS
EPW = E // NW
K = 80
CHUNKS = EPW // K
N_PAD = 10240
RPT = N_PAD // NS

_sc_mesh = plsc.VectorSubcoreMesh(core_axis_name="c", subcore_axis_name="s")


@functools.partial(
    pl.kernel,
    mesh=_sc_mesh,
    out_type=jax.ShapeDtypeStruct((NC, N_PAD, D), jnp.float32),
    scratch_types=[
        pltpu.VMEM((K,), jnp.int32),
        pltpu.VMEM((K,), jnp.int32),
        pltpu.VMEM((K, D), jnp.float32),
        pltpu.VMEM((K, D), jnp.float32),
        pltpu.VMEM_SHARED((N_PAD, D), jnp.float32),
        pltpu.SemaphoreType.DMA,
    ],
)
def _sc_aggregate(x_hbm, src_hbm, dst_hbm, ea_hbm, zeros_hbm, out_hbm,
                  sidx, didx, xrows, erows, acc, sem):
    c = lax.axis_index("c")
    s = lax.axis_index("s")
    wid = c * NS + s
    base = wid * EPW

    pltpu.sync_copy(zeros_hbm.at[pl.ds(s * RPT, RPT)],
                    acc.at[pl.ds(s * RPT, RPT)])
    plsc.subcore_barrier()

    def chunk(j, carry):
        off = base + j * K
        pltpu.sync_copy(src_hbm.at[pl.ds(off, K)], sidx)
        pltpu.sync_copy(dst_hbm.at[pl.ds(off, K)], didx)
        pltpu.async_copy(x_hbm.at[sidx], xrows, sem).wait()
        pltpu.sync_copy(ea_hbm.at[pl.ds(off, K), :], erows)

        def row(i, rcarry):
            for cc in range(D // 16):
                sl = pl.ds(cc * 16, 16)
                v = xrows[i, sl] + erows[i, sl]
                xrows[i, sl] = jnp.maximum(v, 0.0)
            return rcarry

        lax.fori_loop(0, K, row, 0)
        pltpu.sync_copy(xrows, acc.at[didx], add=True)
        return carry

    lax.fori_loop(0, CHUNKS, chunk, 0)

    plsc.subcore_barrier()
    pltpu.sync_copy(acc.at[pl.ds(s * RPT, RPT)],
                    out_hbm.at[c, pl.ds(s * RPT, RPT)])


def _dense_body(x_ref, p_ref, w1_ref, b1_ref, w2_ref, b2_ref, o_ref):
    x = x_ref[...]
    h = x + p_ref[0, :N] + p_ref[1, :N]
    h1 = jnp.maximum(
        jnp.dot(h, w1_ref[...], preferred_element_type=jnp.float32)
        + b1_ref[...], 0.0)
    h2 = (jnp.dot(h1, w2_ref[...], preferred_element_type=jnp.float32)
          + b2_ref[...])
    y = x + h2
    mean = jnp.mean(y, axis=0, keepdims=True)
    var = jnp.mean((y - mean) ** 2, axis=0, keepdims=True)
    o_ref[...] = (y - mean) * lax.rsqrt(var + 1e-5)


def kernel(x, edge_index, edge_attr, W1, b1, W2, b2):
    zeros = jnp.zeros((N_PAD, D), jnp.float32)
    partials = _sc_aggregate(x, edge_index[0], edge_index[1], edge_attr, zeros)
    out = pl.pallas_call(
        _dense_body,
        out_shape=jax.ShapeDtypeStruct((N, D), jnp.float32),
    )(x, partials, W1, b1.reshape(1, D), W2, b2.reshape(1, D))
    return out


# speedup vs baseline: 5.1719x; 1.5346x over previous
import functools

import jax
import jax.numpy as jnp
from jax import lax
from jax.experimental import pallas as pl
from jax.experimental.pallas import tpu as pltpu
from jax.experimental.pallas import tpu_sc as plsc

N = 10000
E = 320000
D = 128

NC = 2
NS = 16
NW = NC * NS
EPW = E // NW
K = 40
CHUNKS = EPW // K
N_PAD = 10240
RPT = N_PAD // NS

NBUF = 4
PD = 2

_sc_mesh = plsc.VectorSubcoreMesh(core_axis_name="c", subcore_axis_name="s")

_scratch = []
for _ in range(NBUF):
    _scratch += [pltpu.VMEM((K,), jnp.int32),
                 pltpu.VMEM((K,), jnp.int32),
                 pltpu.VMEM((K, D), jnp.float32),
                 pltpu.VMEM((K, D), jnp.float32)]
_scratch += [pltpu.SemaphoreType.DMA] * (3 * NBUF)
_scratch += [pltpu.VMEM_SHARED((N_PAD, D), jnp.float32)]


@functools.partial(
    pl.kernel,
    mesh=_sc_mesh,
    out_type=jax.ShapeDtypeStruct((NC, N_PAD, D), jnp.float32),
    scratch_types=_scratch,
)
def _sc_aggregate(x_hbm, src_hbm, dst_hbm, ea_hbm, zeros_hbm, out_hbm,
                  *refs):
    sidx = [refs[4 * b + 0] for b in range(NBUF)]
    didx = [refs[4 * b + 1] for b in range(NBUF)]
    xr = [refs[4 * b + 2] for b in range(NBUF)]
    er = [refs[4 * b + 3] for b in range(NBUF)]
    gsem = list(refs[4 * NBUF:5 * NBUF])
    esem = list(refs[5 * NBUF:6 * NBUF])
    ssem = list(refs[6 * NBUF:7 * NBUF])
    acc = refs[7 * NBUF]

    c = lax.axis_index("c")
    s = lax.axis_index("s")
    wid = c * NS + s
    base = wid * EPW

    pltpu.sync_copy(zeros_hbm.at[pl.ds(s * RPT, RPT)],
                    acc.at[pl.ds(s * RPT, RPT)])
    plsc.subcore_barrier()

    def issue(b, off):
        pltpu.sync_copy(src_hbm.at[pl.ds(off, K)], sidx[b])
        pltpu.sync_copy(dst_hbm.at[pl.ds(off, K)], didx[b])
        pltpu.async_copy(x_hbm.at[sidx[b]], xr[b], gsem[b])
        pltpu.async_copy(ea_hbm.at[pl.ds(off, K), :], er[b], esem[b])

    def wait_gather(b):
        pltpu.make_async_copy(x_hbm.at[sidx[b]], xr[b], gsem[b]).wait()
        pltpu.make_async_copy(ea_hbm.at[pl.ds(0, K), :], er[b], esem[b]).wait()

    def wait_scatter(b):
        pltpu.make_async_copy(xr[b], acc.at[didx[b]], ssem[b]).wait()

    def process(b):
        wait_gather(b)

        def row(i, rcarry):
            for cc in range(D // 16):
                sl = pl.ds(cc * 16, 16)
                v = xr[b][i, sl] + er[b][i, sl]
                xr[b][i, sl] = jnp.maximum(v, 0.0)
            return rcarry

        lax.fori_loop(0, K, row, 0)
        pltpu.async_copy(xr[b], acc.at[didx[b]], ssem[b], add=True)

    for j in range(PD):
        issue(j % NBUF, base + j * K)
    for j in range(NBUF - PD):
        process(j % NBUF)
        issue((j + PD) % NBUF, base + (j + PD) * K)
    steady0 = NBUF - PD
    nsteady = CHUNKS - PD - steady0
    ngroups = nsteady // NBUF
    rem = nsteady - ngroups * NBUF

    def group(t, carry):
        for bi in range(NBUF):
            j = steady0 + t * NBUF + bi
            b = (steady0 + bi) % NBUF
            process(b)
            nb = (b + PD) % NBUF
            wait_scatter(nb)
            issue(nb, base + (j + PD) * K)
        return carry

    lax.fori_loop(0, ngroups, group, 0)
    for bi in range(rem):
        j = steady0 + ngroups * NBUF + bi
        b = (steady0 + bi) % NBUF
        process(b)
        nb = (b + PD) % NBUF
        wait_scatter(nb)
        issue(nb, base + (j + PD) * K)
    for j in range(CHUNKS - PD, CHUNKS):
        process(j % NBUF)
    for b in range(NBUF):
        wait_scatter(b)

    plsc.subcore_barrier()
    pltpu.sync_copy(acc.at[pl.ds(s * RPT, RPT)],
                    out_hbm.at[c, pl.ds(s * RPT, RPT)])


def _dense_body(x_ref, p_ref, w1_ref, b1_ref, w2_ref, b2_ref, o_ref):
    x = x_ref[...]
    h = x + p_ref[0, :N] + p_ref[1, :N]
    h1 = jnp.maximum(
        jnp.dot(h, w1_ref[...], preferred_element_type=jnp.float32)
        + b1_ref[...], 0.0)
    h2 = (jnp.dot(h1, w2_ref[...], preferred_element_type=jnp.float32)
          + b2_ref[...])
    y = x + h2
    mean = jnp.mean(y, axis=0, keepdims=True)
    var = jnp.mean((y - mean) ** 2, axis=0, keepdims=True)
    o_ref[...] = (y - mean) * lax.rsqrt(var + 1e-5)


def kernel(x, edge_index, edge_attr, W1, b1, W2, b2):
    zeros = jnp.zeros((N_PAD, D), jnp.float32)
    partials = _sc_aggregate(x, edge_index[0], edge_index[1], edge_attr, zeros)
    out = pl.pallas_call(
        _dense_body,
        out_shape=jax.ShapeDtypeStruct((N, D), jnp.float32),
    )(x, partials, W1, b1.reshape(1, D), W2, b2.reshape(1, D))
    return out


# speedup vs baseline: 7.9661x; 1.5403x over previous
import functools

import jax
import jax.numpy as jnp
from jax import lax
from jax.experimental import pallas as pl
from jax.experimental.pallas import tpu as pltpu
from jax.experimental.pallas import tpu_sc as plsc

N = 10000
E = 320000
D = 128

NC = 2
NS = 16
NW = NC * NS
EPW = E // NW
K = 40
CHUNKS = EPW // K
N_PAD = 10240
RPT = N_PAD // NS

ND = 4
NI = 8
DG = 2
DI = 4

_sc_mesh = plsc.VectorSubcoreMesh(core_axis_name="c", subcore_axis_name="s")

_scratch = []
_scratch += [pltpu.VMEM((K,), jnp.int32)] * NI
_scratch += [pltpu.VMEM((K,), jnp.int32)] * NI
_scratch += [pltpu.VMEM((K, D), jnp.float32)] * ND
_scratch += [pltpu.VMEM((K, D), jnp.float32)] * ND
_scratch += [pltpu.SemaphoreType.DMA] * (2 * NI)
_scratch += [pltpu.SemaphoreType.DMA] * (3 * ND)
_scratch += [pltpu.VMEM_SHARED((N_PAD, D), jnp.float32)]


@functools.partial(
    pl.kernel,
    mesh=_sc_mesh,
    out_type=jax.ShapeDtypeStruct((NC, N_PAD, D), jnp.float32),
    scratch_types=_scratch,
)
def _sc_aggregate(x_hbm, src_hbm, dst_hbm, ea_hbm, zeros_hbm, out_hbm,
                  *refs):
    o = 0
    sidx = list(refs[o:o + NI]); o += NI
    didx = list(refs[o:o + NI]); o += NI
    xr = list(refs[o:o + ND]); o += ND
    er = list(refs[o:o + ND]); o += ND
    isems = list(refs[o:o + NI]); o += NI
    isemd = list(refs[o:o + NI]); o += NI
    gsem = list(refs[o:o + ND]); o += ND
    esem = list(refs[o:o + ND]); o += ND
    ssem = list(refs[o:o + ND]); o += ND
    acc = refs[o]

    c = lax.axis_index("c")
    s = lax.axis_index("s")
    wid = c * NS + s
    base = wid * EPW

    pltpu.sync_copy(zeros_hbm.at[pl.ds(s * RPT, RPT)],
                    acc.at[pl.ds(s * RPT, RPT)])
    plsc.subcore_barrier()

    def idx_start(b8, off):
        pltpu.async_copy(src_hbm.at[pl.ds(off, K)], sidx[b8], isems[b8])
        pltpu.async_copy(dst_hbm.at[pl.ds(off, K)], didx[b8], isemd[b8])

    def gather_start(b4, b8, off):
        pltpu.make_async_copy(src_hbm.at[pl.ds(0, K)], sidx[b8],
                              isems[b8]).wait()
        pltpu.make_async_copy(dst_hbm.at[pl.ds(0, K)], didx[b8],
                              isemd[b8]).wait()
        pltpu.async_copy(x_hbm.at[sidx[b8]], xr[b4], gsem[b4])
        pltpu.async_copy(ea_hbm.at[pl.ds(off, K), :], er[b4], esem[b4])

    def wait_scatter(b4, b8):
        pltpu.make_async_copy(xr[b4], acc.at[didx[b8]], ssem[b4]).wait()

    def process(b4, b8):
        pltpu.make_async_copy(x_hbm.at[sidx[b8]], xr[b4], gsem[b4]).wait()
        pltpu.make_async_copy(ea_hbm.at[pl.ds(0, K), :], er[b4],
                              esem[b4]).wait()

        def row(i, rcarry):
            for cc in range(D // 16):
                sl = pl.ds(cc * 16, 16)
                v = xr[b4][i, sl] + er[b4][i, sl]
                xr[b4][i, sl] = jnp.maximum(v, 0.0)
            return rcarry

        lax.fori_loop(0, K, row, 0)
        pltpu.async_copy(xr[b4], acc.at[didx[b8]], ssem[b4], add=True)

    def step(j_off, jpy):
        process(jpy % ND, jpy % NI)
        if jpy >= 2:
            wait_scatter((jpy - DG) % ND, (jpy - DG) % NI)
        if jpy + DG < CHUNKS:
            gather_start((jpy + DG) % ND, (jpy + DG) % NI,
                         base + (j_off + DG) * K)
        if jpy + DI < CHUNKS:
            idx_start((jpy + DI) % NI, base + (j_off + DI) * K)

    for j in range(DI):
        idx_start(j % NI, base + j * K)
    for j in range(DG):
        gather_start(j % ND, j % NI, base + j * K)

    for j in range(NI):
        step(j, j)

    steady0 = NI
    nsteady = ((CHUNKS - DI - steady0) // NI) * NI
    ngroups = nsteady // NI

    def group(t, carry):
        for bi in range(NI):
            step(steady0 + t * NI + bi, steady0 + bi)
        return carry

    lax.fori_loop(0, ngroups, group, 0)

    for j in range(steady0 + nsteady, CHUNKS):
        step(j, j)

    for j in range(CHUNKS - DG, CHUNKS):
        wait_scatter(j % ND, j % NI)

    plsc.subcore_barrier()
    pltpu.sync_copy(acc.at[pl.ds(s * RPT, RPT)],
                    out_hbm.at[c, pl.ds(s * RPT, RPT)])


def _dense_body(x_ref, p_ref, w1_ref, b1_ref, w2_ref, b2_ref, o_ref):
    x = x_ref[...]
    h = x + p_ref[0, :N] + p_ref[1, :N]
    h1 = jnp.maximum(
        jnp.dot(h, w1_ref[...], preferred_element_type=jnp.float32)
        + b1_ref[...], 0.0)
    h2 = (jnp.dot(h1, w2_ref[...], preferred_element_type=jnp.float32)
          + b2_ref[...])
    y = x + h2
    mean = jnp.mean(y, axis=0, keepdims=True)
    var = jnp.mean((y - mean) ** 2, axis=0, keepdims=True)
    o_ref[...] = (y - mean) * lax.rsqrt(var + 1e-5)


def kernel(x, edge_index, edge_attr, W1, b1, W2, b2):
    zeros = jnp.zeros((N_PAD, D), jnp.float32)
    partials = _sc_aggregate(x, edge_index[0], edge_index[1], edge_attr, zeros)
    out = pl.pallas_call(
        _dense_body,
        out_shape=jax.ShapeDtypeStruct((N, D), jnp.float32),
    )(x, partials, W1, b1.reshape(1, D), W2, b2.reshape(1, D))
    return out


# speedup vs baseline: 8.5096x; 1.0682x over previous
import functools

import jax
import jax.numpy as jnp
from jax import lax
from jax.experimental import pallas as pl
from jax.experimental.pallas import tpu as pltpu
from jax.experimental.pallas import tpu_sc as plsc

N = 10000
E = 320000
D = 128

NC = 2
NS = 16
NW = NC * NS
EPW = E // NW
K = 40
CHUNKS = EPW // K
N_PAD = 10240
RPT = N_PAD // NS

ND = 4
NI = 8
DG = 2
DI = 4

_sc_mesh = plsc.VectorSubcoreMesh(core_axis_name="c", subcore_axis_name="s")

_scratch = []
_scratch += [pltpu.VMEM((K,), jnp.int32)] * NI
_scratch += [pltpu.VMEM((K,), jnp.int32)] * NI
_scratch += [pltpu.VMEM((K, D), jnp.float32)] * ND
_scratch += [pltpu.VMEM((K, D), jnp.float32)] * ND
_scratch += [pltpu.VMEM((K, D), jnp.float32)]
_scratch += [pltpu.SemaphoreType.DMA] * (2 * NI)
_scratch += [pltpu.SemaphoreType.DMA] * (3 * ND)
_scratch += [pltpu.VMEM_SHARED((N_PAD, D), jnp.float32)]


@functools.partial(
    pl.kernel,
    mesh=_sc_mesh,
    out_type=jax.ShapeDtypeStruct((NC, N_PAD, D), jnp.float32),
    scratch_types=_scratch,
)
def _sc_aggregate(x_hbm, ei_hbm, ea_hbm, out_hbm, *refs):
    o = 0
    sidx = list(refs[o:o + NI]); o += NI
    didx = list(refs[o:o + NI]); o += NI
    xr = list(refs[o:o + ND]); o += ND
    er = list(refs[o:o + ND]); o += ND
    zbuf = refs[o]; o += 1
    isems = list(refs[o:o + NI]); o += NI
    isemd = list(refs[o:o + NI]); o += NI
    gsem = list(refs[o:o + ND]); o += ND
    esem = list(refs[o:o + ND]); o += ND
    ssem = list(refs[o:o + ND]); o += ND
    acc = refs[o]

    c = lax.axis_index("c")
    s = lax.axis_index("s")
    wid = c * NS + s
    base = wid * EPW

    def idx_start(b8, off):
        pltpu.async_copy(ei_hbm.at[pl.ds(off, K)], sidx[b8], isems[b8])
        pltpu.async_copy(ei_hbm.at[pl.ds(E + off, K)], didx[b8], isemd[b8])

    def gather_start(b4, b8, off):
        pltpu.make_async_copy(ei_hbm.at[pl.ds(0, K)], sidx[b8],
                              isems[b8]).wait()
        pltpu.make_async_copy(ei_hbm.at[pl.ds(0, K)], didx[b8],
                              isemd[b8]).wait()
        pltpu.async_copy(x_hbm.at[sidx[b8]], xr[b4], gsem[b4])
        pltpu.async_copy(ea_hbm.at[pl.ds(off, K), :], er[b4], esem[b4])

    def wait_scatter(b4, b8):
        pltpu.make_async_copy(xr[b4], acc.at[didx[b8]], ssem[b4]).wait()

    def process(b4, b8):
        pltpu.make_async_copy(x_hbm.at[sidx[b8]], xr[b4], gsem[b4]).wait()
        pltpu.make_async_copy(ea_hbm.at[pl.ds(0, K), :], er[b4],
                              esem[b4]).wait()

        def row(i, rcarry):
            for u in range(2):
                for cc in range(D // 16):
                    sl = pl.ds(cc * 16, 16)
                    v = xr[b4][2 * i + u, sl] + er[b4][2 * i + u, sl]
                    xr[b4][2 * i + u, sl] = jnp.maximum(v, 0.0)
            return rcarry

        lax.fori_loop(0, K // 2, row, 0)
        pltpu.async_copy(xr[b4], acc.at[didx[b8]], ssem[b4], add=True)

    def step(j_off, jpy):
        process(jpy % ND, jpy % NI)
        if jpy >= 2:
            wait_scatter((jpy - DG) % ND, (jpy - DG) % NI)
        if jpy + DG < CHUNKS:
            gather_start((jpy + DG) % ND, (jpy + DG) % NI,
                         base + (j_off + DG) * K)
        if jpy + DI < CHUNKS:
            idx_start((jpy + DI) % NI, base + (j_off + DI) * K)

    for j in range(DI):
        idx_start(j % NI, base + j * K)
    for j in range(DG):
        gather_start(j % ND, j % NI, base + j * K)

    def zrow(i, rcarry):
        zv = jnp.zeros((16,), jnp.float32)
        for cc in range(D // 16):
            zbuf[i, pl.ds(cc * 16, 16)] = zv
        return rcarry

    lax.fori_loop(0, K, zrow, 0)
    for t in range(RPT // K):
        pltpu.sync_copy(zbuf, acc.at[pl.ds(s * RPT + t * K, K)])
    plsc.subcore_barrier()

    for j in range(NI):
        step(j, j)

    steady0 = NI
    nsteady = ((CHUNKS - DI - steady0) // NI) * NI
    ngroups = nsteady // NI

    def group(t, carry):
        for bi in range(NI):
            step(steady0 + t * NI + bi, steady0 + bi)
        return carry

    lax.fori_loop(0, ngroups, group, 0)

    for j in range(steady0 + nsteady, CHUNKS):
        step(j, j)

    for j in range(CHUNKS - DG, CHUNKS):
        wait_scatter(j % ND, j % NI)

    plsc.subcore_barrier()
    pltpu.sync_copy(acc.at[pl.ds(s * RPT, RPT)],
                    out_hbm.at[c, pl.ds(s * RPT, RPT)])


def _dense_body(x_ref, p_ref, w1_ref, b1_ref, w2_ref, b2_ref, o_ref):
    x = x_ref[...]
    h = x + p_ref[0, :N] + p_ref[1, :N]
    h1 = jnp.maximum(
        jnp.dot(h, w1_ref[...], preferred_element_type=jnp.float32)
        + b1_ref[...], 0.0)
    h2 = (jnp.dot(h1, w2_ref[...], preferred_element_type=jnp.float32)
          + b2_ref[...])
    y = x + h2
    mean = jnp.mean(y, axis=0, keepdims=True)
    var = jnp.mean((y - mean) ** 2, axis=0, keepdims=True)
    o_ref[...] = (y - mean) * lax.rsqrt(var + 1e-5)


def kernel(x, edge_index, edge_attr, W1, b1, W2, b2):
    partials = _sc_aggregate(x, edge_index.reshape(2 * E), edge_attr)
    out = pl.pallas_call(
        _dense_body,
        out_shape=jax.ShapeDtypeStruct((N, D), jnp.float32),
    )(x, partials, W1, b1.reshape(1, D), W2, b2.reshape(1, D))
    return out
